# baseline (speedup 1.0000x reference)
import functools

import jax
import jax.numpy as jnp
from jax import lax
from jax.experimental import pallas as pl
from jax.experimental.pallas import tpu as pltpu
from jax.experimental.pallas import tpu_sc as plsc

EMBED_DIM = 64
LANES = 128
NBUF = 4


@functools.lru_cache(maxsize=None)
def _make_sc_gather(n_units: int, d: int):
    info = plsc.get_sparse_core_info()
    nc, ns = info.num_cores, info.num_subcores
    nw = nc * ns
    units_per_w = n_units // nw
    assert units_per_w * nw == n_units
    assert units_per_w % NBUF == 0 and units_per_w // NBUF >= 2
    mesh = plsc.VectorSubcoreMesh(core_axis_name="c", subcore_axis_name="s")

    @functools.partial(
        pl.kernel,
        mesh=mesh,
        out_type=jax.ShapeDtypeStruct((n_units, d, LANES), jnp.float32),
        scratch_types=[
            pltpu.VMEM((NBUF, LANES), jnp.int32),
            pltpu.VMEM((NBUF, LANES, d), jnp.float32),
        ]
        + [pltpu.SemaphoreType.DMA] * (2 * NBUF),
        compiler_params=pltpu.CompilerParams(use_tc_tiling_on_sc=False),
    )
    def sc_gather(table_hbm, idx_hbm, out_hbm, idx_v, rows_v, *sems):
        wid = lax.axis_index("s") * nc + lax.axis_index("c")
        base = wid * units_per_w
        gsems = sems[:NBUF]
        wsems = sems[NBUF:]

        def stage_idx(g, b):
            pltpu.sync_copy(idx_hbm.at[base + g], idx_v.at[b])

        def start_gather(b):
            pltpu.async_copy(table_hbm.at[idx_v.at[b]], rows_v.at[b], gsems[b])

        def wait_gather(b):
            pltpu.make_async_copy(table_hbm.at[idx_v.at[b]],
                                  rows_v.at[b], gsems[b]).wait()

        def start_write(g, b):
            pltpu.async_copy(rows_v.at[b, pl.ds(0, d)],
                             out_hbm.at[base + g, :, pl.ds(0, d)], wsems[b])
            pltpu.async_copy(rows_v.at[b, pl.ds(d, d)],
                             out_hbm.at[base + g, :, pl.ds(d, d)], wsems[b])

        def wait_write(g, b):
            pltpu.make_async_copy(rows_v.at[b, pl.ds(0, d)],
                                  out_hbm.at[base + g, :, pl.ds(0, d)],
                                  wsems[b]).wait()
            pltpu.make_async_copy(rows_v.at[b, pl.ds(d, d)],
                                  out_hbm.at[base + g, :, pl.ds(d, d)],
                                  wsems[b]).wait()

        for b in range(NBUF):
            stage_idx(b, b)
            start_gather(b)

        def outer(t, carry):
            for b in range(NBUF):
                g = NBUF * t + b
                wait_gather(b)
                start_write(g, b)
                stage_idx(g + NBUF, b)
                wait_write(g, b)
                start_gather(b)
            return carry

        lax.fori_loop(0, units_per_w // NBUF - 1, outer, 0)

        for b in range(NBUF):
            g = units_per_w - NBUF + b
            wait_gather(b)
            start_write(g, b)
        for b in range(NBUF):
            g = units_per_w - NBUF + b
            wait_write(g, b)

    return sc_gather


def _tc_transpose_body(in_ref, out_ref):
    y = in_ref[0]
    a = y[:, :EMBED_DIM]
    b = y[:, EMBED_DIM:]
    out_ref[0] = jnp.concatenate(
        [jnp.transpose(a), jnp.transpose(b)], axis=1)


@functools.lru_cache(maxsize=None)
def _make_tc_transpose(hist: int, kblocks: int):
    n_units = hist * kblocks
    grid = (hist, kblocks)
    return pl.pallas_call(
        _tc_transpose_body,
        grid=grid,
        in_specs=[pl.BlockSpec((1, EMBED_DIM, LANES),
                               lambda h, k: (h * grid[1] + k, 0, 0))],
        out_specs=pl.BlockSpec((1, EMBED_DIM, LANES),
                               lambda h, k: (h, 0, k)),
        out_shape=jax.ShapeDtypeStruct(
            (hist, EMBED_DIM, kblocks * LANES), jnp.float32),
        compiler_params=pltpu.CompilerParams(
            dimension_semantics=("arbitrary", "arbitrary")),
    )


def kernel(input_ids, table):
    batch, hist = input_ids.shape
    kblocks = batch // LANES
    n_units = hist * kblocks
    ids_t = input_ids.astype(jnp.int32).T.reshape(n_units, LANES)
    v = _make_sc_gather(n_units, EMBED_DIM)(table, ids_t)
    l = _make_tc_transpose(hist, kblocks)(v)
    return l.transpose(2, 0, 1)


# speedup vs baseline: 8.1254x; 8.1254x over previous
import functools

import jax
import jax.numpy as jnp
from jax import lax
from jax.experimental import pallas as pl
from jax.experimental.pallas import tpu as pltpu
from jax.experimental.pallas import tpu_sc as plsc

EMBED_DIM = 64
LANES = 128
NBUF = 4


@functools.lru_cache(maxsize=None)
def _make_sc_gather(n_units: int, d: int):
    info = plsc.get_sparse_core_info()
    nc, ns = info.num_cores, info.num_subcores
    nw = nc * ns
    units_per_w = n_units // nw
    assert units_per_w * nw == n_units
    assert units_per_w % NBUF == 0 and units_per_w // NBUF >= 2
    mesh = plsc.VectorSubcoreMesh(core_axis_name="c", subcore_axis_name="s")

    @functools.partial(
        pl.kernel,
        mesh=mesh,
        out_type=jax.ShapeDtypeStruct((n_units, d, LANES), jnp.float32),
        scratch_types=[
            pltpu.VMEM((NBUF, LANES), jnp.int32),
            pltpu.VMEM((NBUF, LANES, d), jnp.float32),
        ]
        + [pltpu.SemaphoreType.DMA] * (2 * NBUF),
        compiler_params=pltpu.CompilerParams(use_tc_tiling_on_sc=False),
    )
    def sc_gather(table_hbm, idx_hbm, out_hbm, idx_v, rows_v, *sems):
        wid = lax.axis_index("s") * nc + lax.axis_index("c")
        base = wid * units_per_w
        gsems = sems[:NBUF]
        wsems = sems[NBUF:]

        def stage_idx(g, b):
            pltpu.sync_copy(idx_hbm.at[base + g], idx_v.at[b])

        def start_gather(b):
            pltpu.async_copy(table_hbm.at[idx_v.at[b]], rows_v.at[b], gsems[b])

        def wait_gather(b):
            pltpu.make_async_copy(table_hbm.at[idx_v.at[b]],
                                  rows_v.at[b], gsems[b]).wait()

        def start_write(g, b):
            pltpu.async_copy(rows_v.at[b, pl.ds(0, d)],
                             out_hbm.at[base + g, :, pl.ds(0, d)], wsems[b])
            pltpu.async_copy(rows_v.at[b, pl.ds(d, d)],
                             out_hbm.at[base + g, :, pl.ds(d, d)], wsems[b])

        def wait_write(g, b):
            pltpu.make_async_copy(rows_v.at[b, pl.ds(0, d)],
                                  out_hbm.at[base + g, :, pl.ds(0, d)],
                                  wsems[b]).wait()
            pltpu.make_async_copy(rows_v.at[b, pl.ds(d, d)],
                                  out_hbm.at[base + g, :, pl.ds(d, d)],
                                  wsems[b]).wait()

        for b in range(NBUF):
            stage_idx(b, b)
            start_gather(b)

        def outer(t, carry):
            for b in range(NBUF):
                g = NBUF * t + b
                wait_gather(b)
                start_write(g, b)
                stage_idx(g + NBUF, b)
                wait_write(g, b)
                start_gather(b)
            return carry

        lax.fori_loop(0, units_per_w // NBUF - 1, outer, 0)

        for b in range(NBUF):
            g = units_per_w - NBUF + b
            wait_gather(b)
            start_write(g, b)
        for b in range(NBUF):
            g = units_per_w - NBUF + b
            wait_write(g, b)

    return sc_gather


@functools.lru_cache(maxsize=None)
def _make_tc_transpose(hist: int, kblocks: int):
    def body(in_ref, out_ref):
        y2 = in_ref[...].reshape(kblocks * EMBED_DIM, 2 * EMBED_DIM)
        eye = (lax.broadcasted_iota(jnp.int32, (2 * EMBED_DIM,) * 2, 0)
               == lax.broadcasted_iota(jnp.int32, (2 * EMBED_DIM,) * 2, 1)
               ).astype(jnp.float32)
        yt = lax.dot_general(eye, y2, (((1,), (1,)), ((), ())),
                             preferred_element_type=jnp.float32)
        for u in range(kblocks):
            out_ref[0, :, u * LANES:u * LANES + EMBED_DIM] = (
                yt[:EMBED_DIM, u * EMBED_DIM:(u + 1) * EMBED_DIM])
            out_ref[0, :, u * LANES + EMBED_DIM:(u + 1) * LANES] = (
                yt[EMBED_DIM:, u * EMBED_DIM:(u + 1) * EMBED_DIM])

    return pl.pallas_call(
        body,
        grid=(hist,),
        in_specs=[pl.BlockSpec((kblocks, EMBED_DIM, LANES),
                               lambda h: (h, 0, 0))],
        out_specs=pl.BlockSpec((1, EMBED_DIM, kblocks * LANES),
                               lambda h: (h, 0, 0)),
        out_shape=jax.ShapeDtypeStruct(
            (hist, EMBED_DIM, kblocks * LANES), jnp.float32),
        compiler_params=pltpu.CompilerParams(
            dimension_semantics=("parallel",)),
    )


def kernel(input_ids, table):
    batch, hist = input_ids.shape
    kblocks = batch // LANES
    n_units = hist * kblocks
    ids_t = input_ids.astype(jnp.int32).T.reshape(n_units, LANES)
    v = _make_sc_gather(n_units, EMBED_DIM)(table, ids_t)
    l = _make_tc_transpose(hist, kblocks)(v)
    return l.transpose(2, 0, 1)


# speedup vs baseline: 9.5715x; 1.1780x over previous
import functools

import jax
import jax.numpy as jnp
from jax import lax
from jax.experimental import pallas as pl
from jax.experimental.pallas import tpu as pltpu
from jax.experimental.pallas import tpu_sc as plsc

EMBED_DIM = 64
LANES = 128
NBUF = 4
HB = 2


@functools.lru_cache(maxsize=None)
def _make_sc_gather(n_units: int, d: int):
    info = plsc.get_sparse_core_info()
    nc, ns = info.num_cores, info.num_subcores
    nw = nc * ns
    units_per_w = n_units // nw
    assert units_per_w * nw == n_units
    assert units_per_w % NBUF == 0 and units_per_w // NBUF >= 2
    mesh = plsc.VectorSubcoreMesh(core_axis_name="c", subcore_axis_name="s")

    @functools.partial(
        pl.kernel,
        mesh=mesh,
        out_type=jax.ShapeDtypeStruct((n_units, d, LANES), jnp.float32),
        scratch_types=[
            pltpu.VMEM((NBUF, LANES), jnp.int32),
            pltpu.VMEM((NBUF, LANES, d), jnp.float32),
        ]
        + [pltpu.SemaphoreType.DMA] * (2 * NBUF),
        compiler_params=pltpu.CompilerParams(use_tc_tiling_on_sc=False),
    )
    def sc_gather(table_hbm, idx_hbm, out_hbm, idx_v, rows_v, *sems):
        wid = lax.axis_index("s") * nc + lax.axis_index("c")
        base = wid * units_per_w
        gsems = sems[:NBUF]
        wsems = sems[NBUF:]

        def stage_idx(g, b):
            pltpu.sync_copy(idx_hbm.at[base + g], idx_v.at[b])

        def start_gather(b):
            pltpu.async_copy(table_hbm.at[idx_v.at[b]], rows_v.at[b], gsems[b])

        def wait_gather(b):
            pltpu.make_async_copy(table_hbm.at[idx_v.at[b]],
                                  rows_v.at[b], gsems[b]).wait()

        def start_write(g, b):
            pltpu.async_copy(rows_v.at[b, pl.ds(0, d)],
                             out_hbm.at[base + g, :, pl.ds(0, d)], wsems[b])
            pltpu.async_copy(rows_v.at[b, pl.ds(d, d)],
                             out_hbm.at[base + g, :, pl.ds(d, d)], wsems[b])

        def wait_write(g, b):
            pltpu.make_async_copy(rows_v.at[b, pl.ds(0, d)],
                                  out_hbm.at[base + g, :, pl.ds(0, d)],
                                  wsems[b]).wait()
            pltpu.make_async_copy(rows_v.at[b, pl.ds(d, d)],
                                  out_hbm.at[base + g, :, pl.ds(d, d)],
                                  wsems[b]).wait()

        for b in range(NBUF):
            stage_idx(b, b)
            start_gather(b)

        def outer(t, carry):
            for b in range(NBUF):
                g = NBUF * t + b
                wait_gather(b)
                start_write(g, b)
                stage_idx(g + NBUF, b)
                wait_write(g, b)
                start_gather(b)
            return carry

        lax.fori_loop(0, units_per_w // NBUF - 1, outer, 0)

        for b in range(NBUF):
            g = units_per_w - NBUF + b
            wait_gather(b)
            start_write(g, b)
        for b in range(NBUF):
            g = units_per_w - NBUF + b
            wait_write(g, b)

    return sc_gather


@functools.lru_cache(maxsize=None)
def _make_tc_transpose(hist: int, kblocks: int):
    def body(in_ref, out_ref):
        eye = (lax.broadcasted_iota(jnp.int32, (2 * EMBED_DIM,) * 2, 0)
               == lax.broadcasted_iota(jnp.int32, (2 * EMBED_DIM,) * 2, 1)
               ).astype(jnp.float32)
        for h in range(HB):
            y2 = in_ref[h * kblocks:(h + 1) * kblocks].reshape(
                kblocks * EMBED_DIM, 2 * EMBED_DIM)
            yt = lax.dot_general(eye, y2, (((1,), (1,)), ((), ())),
                                 preferred_element_type=jnp.float32)
            for u in range(kblocks):
                out_ref[h, :, u * LANES:u * LANES + EMBED_DIM] = (
                    yt[:EMBED_DIM, u * EMBED_DIM:(u + 1) * EMBED_DIM])
                out_ref[h, :, u * LANES + EMBED_DIM:(u + 1) * LANES] = (
                    yt[EMBED_DIM:, u * EMBED_DIM:(u + 1) * EMBED_DIM])

    return pl.pallas_call(
        body,
        grid=(hist // HB,),
        in_specs=[pl.BlockSpec((HB * kblocks, EMBED_DIM, LANES),
                               lambda h: (h, 0, 0))],
        out_specs=pl.BlockSpec((HB, EMBED_DIM, kblocks * LANES),
                               lambda h: (h, 0, 0)),
        out_shape=jax.ShapeDtypeStruct(
            (hist, EMBED_DIM, kblocks * LANES), jnp.float32),
        compiler_params=pltpu.CompilerParams(
            dimension_semantics=("parallel",)),
    )


def kernel(input_ids, table):
    batch, hist = input_ids.shape
    kblocks = batch // LANES
    n_units = hist * kblocks
    ids_t = input_ids.astype(jnp.int32).T.reshape(n_units, LANES)
    v = _make_sc_gather(n_units, EMBED_DIM)(table, ids_t)
    l = _make_tc_transpose(hist, kblocks)(v)
    return l.transpose(2, 0, 1)


# speedup vs baseline: 10.3509x; 1.0814x over previous
import functools

import jax
import jax.numpy as jnp
from jax import lax
from jax.experimental import pallas as pl
from jax.experimental.pallas import tpu as pltpu
from jax.experimental.pallas import tpu_sc as plsc

EMBED_DIM = 64
LANES = 128
NBUF = 4
HB = 4


@functools.lru_cache(maxsize=None)
def _make_sc_gather(n_units: int, d: int):
    info = plsc.get_sparse_core_info()
    nc, ns = info.num_cores, info.num_subcores
    nw = nc * ns
    units_per_w = n_units // nw
    assert units_per_w * nw == n_units
    assert units_per_w % NBUF == 0 and units_per_w // NBUF >= 2
    mesh = plsc.VectorSubcoreMesh(core_axis_name="c", subcore_axis_name="s")

    @functools.partial(
        pl.kernel,
        mesh=mesh,
        out_type=jax.ShapeDtypeStruct((n_units, d, LANES), jnp.float32),
        scratch_types=[
            pltpu.VMEM((NBUF, LANES), jnp.int32),
            pltpu.VMEM((NBUF, LANES, d), jnp.float32),
        ]
        + [pltpu.SemaphoreType.DMA] * (2 * NBUF),
        compiler_params=pltpu.CompilerParams(use_tc_tiling_on_sc=False),
    )
    def sc_gather(table_hbm, idx_hbm, out_hbm, idx_v, rows_v, *sems):
        wid = lax.axis_index("s") * nc + lax.axis_index("c")
        base = wid * units_per_w
        gsems = sems[:NBUF]
        wsems = sems[NBUF:]

        def stage_idx(g, b):
            pltpu.sync_copy(idx_hbm.at[base + g], idx_v.at[b])

        def start_gather(b):
            pltpu.async_copy(table_hbm.at[idx_v.at[b]], rows_v.at[b], gsems[b])

        def wait_gather(b):
            pltpu.make_async_copy(table_hbm.at[idx_v.at[b]],
                                  rows_v.at[b], gsems[b]).wait()

        def start_write(g, b):
            pltpu.async_copy(rows_v.at[b, pl.ds(0, d)],
                             out_hbm.at[base + g, :, pl.ds(0, d)], wsems[b])
            pltpu.async_copy(rows_v.at[b, pl.ds(d, d)],
                             out_hbm.at[base + g, :, pl.ds(d, d)], wsems[b])

        def wait_write(g, b):
            pltpu.make_async_copy(rows_v.at[b, pl.ds(0, d)],
                                  out_hbm.at[base + g, :, pl.ds(0, d)],
                                  wsems[b]).wait()
            pltpu.make_async_copy(rows_v.at[b, pl.ds(d, d)],
                                  out_hbm.at[base + g, :, pl.ds(d, d)],
                                  wsems[b]).wait()

        for b in range(NBUF):
            stage_idx(b, b)
            start_gather(b)

        def outer(t, carry):
            for b in range(NBUF):
                g = NBUF * t + b
                wait_gather(b)
                start_write(g, b)
                stage_idx(g + NBUF, b)
                wait_write(g, b)
                start_gather(b)
            return carry

        lax.fori_loop(0, units_per_w // NBUF - 1, outer, 0)

        for b in range(NBUF):
            g = units_per_w - NBUF + b
            wait_gather(b)
            start_write(g, b)
        for b in range(NBUF):
            g = units_per_w - NBUF + b
            wait_write(g, b)

    return sc_gather


@functools.lru_cache(maxsize=None)
def _make_tc_transpose(hist: int, kblocks: int):
    def body(in_ref, out_ref):
        eye = (lax.broadcasted_iota(jnp.int32, (2 * EMBED_DIM,) * 2, 0)
               == lax.broadcasted_iota(jnp.int32, (2 * EMBED_DIM,) * 2, 1)
               ).astype(jnp.float32)
        for h in range(HB):
            y2 = in_ref[h * kblocks:(h + 1) * kblocks].reshape(
                kblocks * EMBED_DIM, 2 * EMBED_DIM)
            yt = lax.dot_general(eye, y2, (((1,), (1,)), ((), ())),
                                 preferred_element_type=jnp.float32)
            for u in range(kblocks):
                out_ref[h, :, u * LANES:u * LANES + EMBED_DIM] = (
                    yt[:EMBED_DIM, u * EMBED_DIM:(u + 1) * EMBED_DIM])
                out_ref[h, :, u * LANES + EMBED_DIM:(u + 1) * LANES] = (
                    yt[EMBED_DIM:, u * EMBED_DIM:(u + 1) * EMBED_DIM])

    return pl.pallas_call(
        body,
        grid=(hist // HB,),
        in_specs=[pl.BlockSpec((HB * kblocks, EMBED_DIM, LANES),
                               lambda h: (h, 0, 0))],
        out_specs=pl.BlockSpec((HB, EMBED_DIM, kblocks * LANES),
                               lambda h: (h, 0, 0)),
        out_shape=jax.ShapeDtypeStruct(
            (hist, EMBED_DIM, kblocks * LANES), jnp.float32),
        compiler_params=pltpu.CompilerParams(
            dimension_semantics=("parallel",)),
    )


def kernel(input_ids, table):
    batch, hist = input_ids.shape
    kblocks = batch // LANES
    n_units = hist * kblocks
    ids_t = input_ids.astype(jnp.int32).T.reshape(n_units, LANES)
    v = _make_sc_gather(n_units, EMBED_DIM)(table, ids_t)
    l = _make_tc_transpose(hist, kblocks)(v)
    return l.transpose(2, 0, 1)


# speedup vs baseline: 10.5101x; 1.0154x over previous
import functools

import jax
import jax.numpy as jnp
from jax import lax
from jax.experimental import pallas as pl
from jax.experimental.pallas import tpu as pltpu
from jax.experimental.pallas import tpu_sc as plsc

EMBED_DIM = 64
LANES = 128
NBUF = 4
HB = 8


@functools.lru_cache(maxsize=None)
def _make_sc_gather(n_units: int, d: int):
    info = plsc.get_sparse_core_info()
    nc, ns = info.num_cores, info.num_subcores
    nw = nc * ns
    units_per_w = n_units // nw
    assert units_per_w * nw == n_units
    assert units_per_w % NBUF == 0 and units_per_w // NBUF >= 2
    mesh = plsc.VectorSubcoreMesh(core_axis_name="c", subcore_axis_name="s")

    @functools.partial(
        pl.kernel,
        mesh=mesh,
        out_type=jax.ShapeDtypeStruct((n_units, d, LANES), jnp.float32),
        scratch_types=[
            pltpu.VMEM((NBUF, LANES), jnp.int32),
            pltpu.VMEM((NBUF, LANES, d), jnp.float32),
        ]
        + [pltpu.SemaphoreType.DMA] * (2 * NBUF),
        compiler_params=pltpu.CompilerParams(use_tc_tiling_on_sc=False),
    )
    def sc_gather(table_hbm, idx_hbm, out_hbm, idx_v, rows_v, *sems):
        wid = lax.axis_index("s") * nc + lax.axis_index("c")
        base = wid * units_per_w
        gsems = sems[:NBUF]
        wsems = sems[NBUF:]

        def stage_idx(g, b):
            pltpu.sync_copy(idx_hbm.at[base + g], idx_v.at[b])

        def start_gather(b):
            pltpu.async_copy(table_hbm.at[idx_v.at[b]], rows_v.at[b], gsems[b])

        def wait_gather(b):
            pltpu.make_async_copy(table_hbm.at[idx_v.at[b]],
                                  rows_v.at[b], gsems[b]).wait()

        def start_write(g, b):
            pltpu.async_copy(rows_v.at[b, pl.ds(0, d)],
                             out_hbm.at[base + g, :, pl.ds(0, d)], wsems[b])
            pltpu.async_copy(rows_v.at[b, pl.ds(d, d)],
                             out_hbm.at[base + g, :, pl.ds(d, d)], wsems[b])

        def wait_write(g, b):
            pltpu.make_async_copy(rows_v.at[b, pl.ds(0, d)],
                                  out_hbm.at[base + g, :, pl.ds(0, d)],
                                  wsems[b]).wait()
            pltpu.make_async_copy(rows_v.at[b, pl.ds(d, d)],
                                  out_hbm.at[base + g, :, pl.ds(d, d)],
                                  wsems[b]).wait()

        for b in range(NBUF):
            stage_idx(b, b)
            start_gather(b)

        def outer(t, carry):
            for b in range(NBUF):
                g = NBUF * t + b
                wait_gather(b)
                start_write(g, b)
                stage_idx(g + NBUF, b)
                wait_write(g, b)
                start_gather(b)
            return carry

        lax.fori_loop(0, units_per_w // NBUF - 1, outer, 0)

        for b in range(NBUF):
            g = units_per_w - NBUF + b
            wait_gather(b)
            start_write(g, b)
        for b in range(NBUF):
            g = units_per_w - NBUF + b
            wait_write(g, b)

    return sc_gather


@functools.lru_cache(maxsize=None)
def _make_tc_transpose(hist: int, kblocks: int):
    def body(in_ref, out_ref):
        eye = (lax.broadcasted_iota(jnp.int32, (2 * EMBED_DIM,) * 2, 0)
               == lax.broadcasted_iota(jnp.int32, (2 * EMBED_DIM,) * 2, 1)
               ).astype(jnp.float32)
        for h in range(HB):
            y2 = in_ref[h * kblocks:(h + 1) * kblocks].reshape(
                kblocks * EMBED_DIM, 2 * EMBED_DIM)
            yt = lax.dot_general(eye, y2, (((1,), (1,)), ((), ())),
                                 preferred_element_type=jnp.float32)
            for u in range(kblocks):
                out_ref[h, :, u * LANES:u * LANES + EMBED_DIM] = (
                    yt[:EMBED_DIM, u * EMBED_DIM:(u + 1) * EMBED_DIM])
                out_ref[h, :, u * LANES + EMBED_DIM:(u + 1) * LANES] = (
                    yt[EMBED_DIM:, u * EMBED_DIM:(u + 1) * EMBED_DIM])

    return pl.pallas_call(
        body,
        grid=(hist // HB,),
        in_specs=[pl.BlockSpec((HB * kblocks, EMBED_DIM, LANES),
                               lambda h: (h, 0, 0))],
        out_specs=pl.BlockSpec((HB, EMBED_DIM, kblocks * LANES),
                               lambda h: (h, 0, 0)),
        out_shape=jax.ShapeDtypeStruct(
            (hist, EMBED_DIM, kblocks * LANES), jnp.float32),
        compiler_params=pltpu.CompilerParams(
            dimension_semantics=("parallel",)),
    )


def kernel(input_ids, table):
    batch, hist = input_ids.shape
    kblocks = batch // LANES
    n_units = hist * kblocks
    ids_t = input_ids.astype(jnp.int32).T.reshape(n_units, LANES)
    v = _make_sc_gather(n_units, EMBED_DIM)(table, ids_t)
    l = _make_tc_transpose(hist, kblocks)(v)
    return l.transpose(2, 0, 1)


# speedup vs baseline: 10.7494x; 1.0228x over previous
import functools

import jax
import jax.numpy as jnp
from jax import lax
from jax.experimental import pallas as pl
from jax.experimental.pallas import tpu as pltpu
from jax.experimental.pallas import tpu_sc as plsc

EMBED_DIM = 64
LANES = 128
NBUF = 4
HB = 8


@functools.lru_cache(maxsize=None)
def _make_sc_gather(n_units: int, d: int):
    info = plsc.get_sparse_core_info()
    nc, ns = info.num_cores, info.num_subcores
    nw = nc * ns
    units_per_w = n_units // nw
    assert units_per_w * nw == n_units
    assert units_per_w % NBUF == 0 and units_per_w // NBUF >= 2
    mesh = plsc.VectorSubcoreMesh(core_axis_name="c", subcore_axis_name="s")

    @functools.partial(
        pl.kernel,
        mesh=mesh,
        out_type=jax.ShapeDtypeStruct((n_units, d, LANES), jnp.float32),
        scratch_types=[
            pltpu.VMEM((units_per_w, LANES), jnp.int32),
            pltpu.VMEM((NBUF, LANES, d), jnp.float32),
        ]
        + [pltpu.SemaphoreType.DMA] * (2 * NBUF),
        compiler_params=pltpu.CompilerParams(use_tc_tiling_on_sc=False),
    )
    def sc_gather(table_hbm, idx_hbm, out_hbm, idx_v, rows_v, *sems):
        wid = lax.axis_index("s") * nc + lax.axis_index("c")
        base = wid * units_per_w
        gsems = sems[:NBUF]
        wsems = sems[NBUF:]

        pltpu.sync_copy(idx_hbm.at[pl.ds(base, units_per_w)], idx_v)

        def start_gather(g, b):
            pltpu.async_copy(table_hbm.at[idx_v.at[g]], rows_v.at[b], gsems[b])

        def wait_gather(g, b):
            pltpu.make_async_copy(table_hbm.at[idx_v.at[g]],
                                  rows_v.at[b], gsems[b]).wait()

        def start_write(g, b):
            pltpu.async_copy(rows_v.at[b, pl.ds(0, d)],
                             out_hbm.at[base + g, :, pl.ds(0, d)], wsems[b])
            pltpu.async_copy(rows_v.at[b, pl.ds(d, d)],
                             out_hbm.at[base + g, :, pl.ds(d, d)], wsems[b])

        def wait_write(g, b):
            pltpu.make_async_copy(rows_v.at[b, pl.ds(0, d)],
                                  out_hbm.at[base + g, :, pl.ds(0, d)],
                                  wsems[b]).wait()
            pltpu.make_async_copy(rows_v.at[b, pl.ds(d, d)],
                                  out_hbm.at[base + g, :, pl.ds(d, d)],
                                  wsems[b]).wait()

        for b in range(NBUF):
            start_gather(b, b)

        def outer(t, carry):
            for b in range(NBUF):
                g = NBUF * t + b
                wait_gather(g, b)
                start_write(g, b)
                wait_write(g, b)
                start_gather(g + NBUF, b)
            return carry

        lax.fori_loop(0, units_per_w // NBUF - 1, outer, 0)

        for b in range(NBUF):
            g = units_per_w - NBUF + b
            wait_gather(g, b)
            start_write(g, b)
        for b in range(NBUF):
            g = units_per_w - NBUF + b
            wait_write(g, b)

    return sc_gather


@functools.lru_cache(maxsize=None)
def _make_tc_transpose(hist: int, kblocks: int):
    def body(in_ref, out_ref):
        eye = (lax.broadcasted_iota(jnp.int32, (2 * EMBED_DIM,) * 2, 0)
               == lax.broadcasted_iota(jnp.int32, (2 * EMBED_DIM,) * 2, 1)
               ).astype(jnp.float32)
        for h in range(HB):
            y2 = in_ref[h * kblocks:(h + 1) * kblocks].reshape(
                kblocks * EMBED_DIM, 2 * EMBED_DIM)
            yt = lax.dot_general(eye, y2, (((1,), (1,)), ((), ())),
                                 preferred_element_type=jnp.float32)
            for u in range(kblocks):
                out_ref[h, :, u * LANES:u * LANES + EMBED_DIM] = (
                    yt[:EMBED_DIM, u * EMBED_DIM:(u + 1) * EMBED_DIM])
                out_ref[h, :, u * LANES + EMBED_DIM:(u + 1) * LANES] = (
                    yt[EMBED_DIM:, u * EMBED_DIM:(u + 1) * EMBED_DIM])

    return pl.pallas_call(
        body,
        grid=(hist // HB,),
        in_specs=[pl.BlockSpec((HB * kblocks, EMBED_DIM, LANES),
                               lambda h: (h, 0, 0))],
        out_specs=pl.BlockSpec((HB, EMBED_DIM, kblocks * LANES),
                               lambda h: (h, 0, 0)),
        out_shape=jax.ShapeDtypeStruct(
            (hist, EMBED_DIM, kblocks * LANES), jnp.float32),
        compiler_params=pltpu.CompilerParams(
            dimension_semantics=("parallel",)),
    )


def kernel(input_ids, table):
    batch, hist = input_ids.shape
    kblocks = batch // LANES
    n_units = hist * kblocks
    ids_t = input_ids.astype(jnp.int32).T.reshape(n_units, LANES)
    v = _make_sc_gather(n_units, EMBED_DIM)(table, ids_t)
    l = _make_tc_transpose(hist, kblocks)(v)
    return l.transpose(2, 0, 1)
